# baseline (speedup 1.0000x reference)
import functools

import jax
import jax.numpy as jnp
from jax import lax
from jax.experimental import pallas as pl
from jax.experimental.pallas import tpu as pltpu
from jax.experimental.pallas import tpu_sc as plsc

_BS = 16384
_EMB = 16
_H1 = 128
_H2 = 128

_info = plsc.get_sparse_core_info()
_NC = _info.num_cores
_NS = _info.num_subcores
_NW = _NC * _NS
_BPW = _BS // _NW


def _sc_gather(idx, emb2, emb1):
    mesh = plsc.VectorSubcoreMesh(core_axis_name="c", subcore_axis_name="s")

    @functools.partial(
        pl.kernel,
        mesh=mesh,
        compiler_params=pltpu.CompilerParams(use_tc_tiling_on_sc=False),
        out_type=(
            jax.ShapeDtypeStruct((_BS, _EMB), jnp.float32),
            jax.ShapeDtypeStruct((_BS, 1), jnp.float32),
        ),
        scratch_types=[
            pltpu.VMEM((_BPW,), jnp.int32),
            pltpu.VMEM((_BPW, _EMB), jnp.float32),
            pltpu.VMEM((_BPW, 1), jnp.float32),
            pltpu.SemaphoreType.DMA,
            pltpu.SemaphoreType.DMA,
        ],
    )
    def gather_kernel(idx_hbm, emb2_hbm, emb1_hbm, e_out, f1_out,
                      idx_v, rows_v, f1_v, sem2, sem1):
        wid = lax.axis_index("s") * _NC + lax.axis_index("c")
        base = wid * _BPW
        pltpu.sync_copy(idx_hbm.at[pl.ds(base, _BPW)], idx_v)
        cp2 = pltpu.async_copy(emb2_hbm.at[idx_v], rows_v, sem2)
        cp1 = pltpu.async_copy(emb1_hbm.at[idx_v], f1_v, sem1)
        cp2.wait()
        cp1.wait()
        pltpu.sync_copy(rows_v, e_out.at[pl.ds(base, _BPW)])
        pltpu.sync_copy(f1_v, f1_out.at[pl.ds(base, _BPW)])

    return gather_kernel(idx, emb2, emb1)


def _tc_body(e_ref, f1_ref, w1_ref, c1_ref, w2_ref, c2_ref, wd_ref, cd_ref,
             o_ref):
    e = e_ref[...]
    fm2 = jnp.sum(e * e, axis=1, keepdims=True)
    h1 = jnp.maximum(
        jnp.dot(e, w1_ref[...], preferred_element_type=jnp.float32)
        + c1_ref[...], 0.0)
    h2 = jnp.maximum(
        jnp.dot(h1, w2_ref[...], preferred_element_type=jnp.float32)
        + c2_ref[...], 0.0)
    d = jnp.dot(h2, wd_ref[...], preferred_element_type=jnp.float32)
    z = f1_ref[...] + fm2 + d + cd_ref[...]
    o_ref[...] = 1.0 / (1.0 + jnp.exp(-z))


def _tc_forward(e, f1, w1, c1, w2, c2, wd, cd):
    blk = 2048
    grid = (_BS // blk,)
    return pl.pallas_call(
        _tc_body,
        grid=grid,
        in_specs=[
            pl.BlockSpec((blk, _EMB), lambda i: (i, 0)),
            pl.BlockSpec((blk, 1), lambda i: (i, 0)),
            pl.BlockSpec((_EMB, _H1), lambda i: (0, 0)),
            pl.BlockSpec((1, _H1), lambda i: (0, 0)),
            pl.BlockSpec((_H1, _H2), lambda i: (0, 0)),
            pl.BlockSpec((1, _H2), lambda i: (0, 0)),
            pl.BlockSpec((_H2, 1), lambda i: (0, 0)),
            pl.BlockSpec((1, 1), lambda i: (0, 0)),
        ],
        out_specs=pl.BlockSpec((blk, 1), lambda i: (i, 0)),
        out_shape=jax.ShapeDtypeStruct((_BS, 1), jnp.float32),
    )(e, f1, w1, c1, w2, c2, wd, cd)


def kernel(X_sparse, emb1, emb2, W1, b1, g1, be1, rm1, rv1,
           W2, b2, g2, be2, rm2, rv2, Wd, bd):
    idx = X_sparse.reshape(-1).astype(jnp.int32)
    s1 = g1 / jnp.sqrt(rv1 + 1e-5)
    w1 = W1 * s1[None, :]
    c1 = ((b1 - rm1) * s1 + be1)[None, :]
    s2 = g2 / jnp.sqrt(rv2 + 1e-5)
    w2 = W2 * s2[None, :]
    c2 = ((b2 - rm2) * s2 + be2)[None, :]
    cd = bd[None, :]

    e, f1 = _sc_gather(idx, emb2, emb1)
    return _tc_forward(e, f1, w1, c1, w2, c2, Wd, cd)


# speedup vs baseline: 2.6122x; 2.6122x over previous
import functools

import jax
import jax.numpy as jnp
from jax import lax
from jax.experimental import pallas as pl
from jax.experimental.pallas import tpu as pltpu
from jax.experimental.pallas import tpu_sc as plsc

_BS = 16384
_VOCAB = 1000000
_EMB = 16
_H1 = 128
_H2 = 128

_info = plsc.get_sparse_core_info()
_NC = _info.num_cores
_NS = _info.num_subcores
_NW = _NC * _NS
_BPW = _BS // _NW
_CH = 128
_NCHUNK = _BPW // _CH

_E1ROWS = (_VOCAB + 127) // 128


def _sc_gather(idx, p_tbl, e1p):
    mesh = plsc.VectorSubcoreMesh(core_axis_name="c", subcore_axis_name="s")

    @functools.partial(
        pl.kernel,
        mesh=mesh,
        compiler_params=pltpu.CompilerParams(
            use_tc_tiling_on_sc=False, needs_layout_passes=False),
        out_type=(
            jax.ShapeDtypeStruct((_EMB, _BS), jnp.float32),
            jax.ShapeDtypeStruct((_BS,), jnp.float32),
        ),
        scratch_types=[
            pltpu.VMEM((_BPW,), jnp.int32),
            pltpu.VMEM((_CH,), jnp.int32),
            pltpu.VMEM((_CH,), jnp.int32),
            pltpu.VMEM((_CH, 128), jnp.float32),
            pltpu.VMEM((_CH, 128), jnp.float32),
            pltpu.VMEM((_EMB, _BPW), jnp.float32),
            pltpu.VMEM((_BPW,), jnp.float32),
            pltpu.SemaphoreType.DMA,
            pltpu.SemaphoreType.DMA,
        ],
    )
    def gather_kernel(idx_hbm, p_hbm, e1p_hbm, et_out, f1_out,
                      idx_v, rv2_v, rvf_v, rows_v, f1rows_v,
                      cols_v, f1c_v, sem_a, sem_b):
        wid = lax.axis_index("s") * _NC + lax.axis_index("c")
        base = wid * _BPW
        pltpu.sync_copy(idx_hbm.at[pl.ds(base, _BPW)], idx_v)

        def chunk_body(h, carry):
            off = h * _CH

            def rows_body(j, c):
                v16 = idx_v[pl.ds(off + j * 16, 16)]
                rv2_v[pl.ds(j * 16, 16)] = lax.shift_right_logical(v16, 3)
                rvf_v[pl.ds(j * 16, 16)] = lax.shift_right_logical(v16, 7)
                return c

            lax.fori_loop(0, _CH // 16, rows_body, 0)

            cp_a = pltpu.async_copy(p_hbm.at[rv2_v], rows_v, sem_a)
            cp_b = pltpu.async_copy(e1p_hbm.at[rvf_v], f1rows_v, sem_b)
            cp_a.wait()
            cp_b.wait()

            def sel_body(g, c):
                i0 = g * 16
                v16 = idx_v[pl.ds(off + i0, 16)]
                row = i0 + lax.iota(jnp.int32, 16)
                lane0 = (v16 & 7) * 16
                for j in range(_EMB):
                    vec = plsc.load_gather(rows_v, [row, lane0 + j])
                    cols_v[j, pl.ds(off + i0, 16)] = vec
                f1vec = plsc.load_gather(f1rows_v, [row, v16 & 127])
                f1c_v[pl.ds(off + i0, 16)] = f1vec
                return c

            lax.fori_loop(0, _CH // 16, sel_body, 0)
            return carry

        lax.fori_loop(0, _NCHUNK, chunk_body, 0)

        pltpu.sync_copy(cols_v, et_out.at[:, pl.ds(base, _BPW)])
        pltpu.sync_copy(f1c_v, f1_out.at[pl.ds(base, _BPW)])

    return gather_kernel(idx, p_tbl, e1p)


def _tc_body(et_ref, f1_ref, w1t_ref, c1_ref, w2t_ref, c2_ref, wdt_ref,
             cd_ref, o_ref):
    et = et_ref[...]
    fm2 = jnp.sum(et * et, axis=0, keepdims=True)
    h1 = jnp.maximum(
        jnp.dot(w1t_ref[...], et, preferred_element_type=jnp.float32)
        + c1_ref[...], 0.0)
    h2 = jnp.maximum(
        jnp.dot(w2t_ref[...], h1, preferred_element_type=jnp.float32)
        + c2_ref[...], 0.0)
    d = jnp.dot(wdt_ref[...], h2, preferred_element_type=jnp.float32)
    z = f1_ref[...] + fm2 + d + cd_ref[...]
    o_ref[...] = 1.0 / (1.0 + jnp.exp(-z))


def _tc_forward(et, f1, w1t, c1, w2t, c2, wdt, cd):
    blk = 2048
    grid = (_BS // blk,)
    return pl.pallas_call(
        _tc_body,
        grid=grid,
        in_specs=[
            pl.BlockSpec((_EMB, blk), lambda i: (0, i)),
            pl.BlockSpec((1, blk), lambda i: (0, i)),
            pl.BlockSpec((_H1, _EMB), lambda i: (0, 0)),
            pl.BlockSpec((_H1, 1), lambda i: (0, 0)),
            pl.BlockSpec((_H2, _H1), lambda i: (0, 0)),
            pl.BlockSpec((_H2, 1), lambda i: (0, 0)),
            pl.BlockSpec((1, _H2), lambda i: (0, 0)),
            pl.BlockSpec((1, 1), lambda i: (0, 0)),
        ],
        out_specs=pl.BlockSpec((1, blk), lambda i: (0, i)),
        out_shape=jax.ShapeDtypeStruct((1, _BS), jnp.float32),
    )(et, f1, w1t, c1, w2t, c2, wdt, cd)


def kernel(X_sparse, emb1, emb2, W1, b1, g1, be1, rm1, rv1,
           W2, b2, g2, be2, rm2, rv2, Wd, bd):
    idx = X_sparse.reshape(-1).astype(jnp.int32)
    p_tbl = emb2.reshape(_VOCAB // 8, 128)
    e1p = jnp.pad(emb1.reshape(-1), (0, _E1ROWS * 128 - _VOCAB)).reshape(
        _E1ROWS, 128)
    s1 = g1 / jnp.sqrt(rv1 + 1e-5)
    w1t = (W1 * s1[None, :]).T
    c1 = ((b1 - rm1) * s1 + be1)[:, None]
    s2 = g2 / jnp.sqrt(rv2 + 1e-5)
    w2t = (W2 * s2[None, :]).T
    c2 = ((b2 - rm2) * s2 + be2)[:, None]
    wdt = Wd.T
    cd = bd[None, :]

    et, f1 = _sc_gather(idx, p_tbl, e1p)
    out = _tc_forward(et, f1.reshape(1, _BS), w1t, c1, w2t, c2, wdt, cd)
    return out.reshape(_BS, 1)


# speedup vs baseline: 2.6266x; 1.0055x over previous
import functools

import jax
import jax.numpy as jnp
from jax import lax
from jax.experimental import pallas as pl
from jax.experimental.pallas import tpu as pltpu
from jax.experimental.pallas import tpu_sc as plsc

_BS = 16384
_VOCAB = 1000000
_EMB = 16
_H1 = 128
_H2 = 128

_info = plsc.get_sparse_core_info()
_NC = _info.num_cores
_NS = _info.num_subcores
_NW = _NC * _NS
_BPW = _BS // _NW
_CH = 128
_NCHUNK = _BPW // _CH


def _sc_gather(idx, p_tbl, e1p):
    mesh = plsc.VectorSubcoreMesh(core_axis_name="c", subcore_axis_name="s")

    @functools.partial(
        pl.kernel,
        mesh=mesh,
        compiler_params=pltpu.CompilerParams(
            use_tc_tiling_on_sc=False, needs_layout_passes=False),
        out_type=(
            jax.ShapeDtypeStruct((_EMB, _BS), jnp.float32),
            jax.ShapeDtypeStruct((_BS,), jnp.float32),
        ),
        scratch_types=[
            pltpu.VMEM((_BPW,), jnp.int32),
            pltpu.VMEM((_CH,), jnp.int32),
            pltpu.VMEM((_CH,), jnp.int32),
            pltpu.VMEM((_CH, 128), jnp.float32),
            pltpu.VMEM((_CH, 16), jnp.float32),
            pltpu.VMEM((_EMB, _BPW), jnp.float32),
            pltpu.VMEM((_BPW,), jnp.float32),
            pltpu.SemaphoreType.DMA,
            pltpu.SemaphoreType.DMA,
        ],
    )
    def gather_kernel(idx_hbm, p_hbm, e1p_hbm, et_out, f1_out,
                      idx_v, rv2_v, rvf_v, rows_v, f1rows_v,
                      cols_v, f1c_v, sem_a, sem_b):
        wid = lax.axis_index("s") * _NC + lax.axis_index("c")
        base = wid * _BPW
        pltpu.sync_copy(idx_hbm.at[pl.ds(base, _BPW)], idx_v)

        def chunk_body(h, carry):
            off = h * _CH

            def rows_body(j, c):
                v16 = idx_v[pl.ds(off + j * 16, 16)]
                rv2_v[pl.ds(j * 16, 16)] = lax.shift_right_logical(v16, 3)
                rvf_v[pl.ds(j * 16, 16)] = lax.shift_right_logical(v16, 4)
                return c

            lax.fori_loop(0, _CH // 16, rows_body, 0)

            cp_a = pltpu.async_copy(p_hbm.at[rv2_v], rows_v, sem_a)
            cp_b = pltpu.async_copy(e1p_hbm.at[rvf_v], f1rows_v, sem_b)
            cp_a.wait()
            cp_b.wait()

            def sel_body(g, c):
                i0 = g * 16
                v16 = idx_v[pl.ds(off + i0, 16)]
                row = i0 + lax.iota(jnp.int32, 16)
                lane0 = (v16 & 7) * 16
                for j in range(_EMB):
                    vec = plsc.load_gather(rows_v, [row, lane0 + j])
                    cols_v[j, pl.ds(off + i0, 16)] = vec
                f1vec = plsc.load_gather(f1rows_v, [row, v16 & 15])
                f1c_v[pl.ds(off + i0, 16)] = f1vec
                return c

            lax.fori_loop(0, _CH // 16, sel_body, 0)
            return carry

        lax.fori_loop(0, _NCHUNK, chunk_body, 0)

        pltpu.sync_copy(cols_v, et_out.at[:, pl.ds(base, _BPW)])
        pltpu.sync_copy(f1c_v, f1_out.at[pl.ds(base, _BPW)])

    return gather_kernel(idx, p_tbl, e1p)


def _tc_body(et_ref, f1_ref, w1t_ref, c1_ref, w2t_ref, c2_ref, wdt_ref,
             cd_ref, o_ref):
    et = et_ref[...]
    fm2 = jnp.sum(et * et, axis=0, keepdims=True)
    h1 = jnp.maximum(
        jnp.dot(w1t_ref[...], et, preferred_element_type=jnp.float32)
        + c1_ref[...], 0.0)
    h2 = jnp.maximum(
        jnp.dot(w2t_ref[...], h1, preferred_element_type=jnp.float32)
        + c2_ref[...], 0.0)
    d = jnp.dot(wdt_ref[...], h2, preferred_element_type=jnp.float32)
    z = f1_ref[...] + fm2 + d + cd_ref[...]
    o_ref[...] = 1.0 / (1.0 + jnp.exp(-z))


def _tc_forward(et, f1, w1t, c1, w2t, c2, wdt, cd):
    blk = 2048
    grid = (_BS // blk,)
    return pl.pallas_call(
        _tc_body,
        grid=grid,
        in_specs=[
            pl.BlockSpec((_EMB, blk), lambda i: (0, i)),
            pl.BlockSpec((1, blk), lambda i: (0, i)),
            pl.BlockSpec((_H1, _EMB), lambda i: (0, 0)),
            pl.BlockSpec((_H1, 1), lambda i: (0, 0)),
            pl.BlockSpec((_H2, _H1), lambda i: (0, 0)),
            pl.BlockSpec((_H2, 1), lambda i: (0, 0)),
            pl.BlockSpec((1, _H2), lambda i: (0, 0)),
            pl.BlockSpec((1, 1), lambda i: (0, 0)),
        ],
        out_specs=pl.BlockSpec((1, blk), lambda i: (0, i)),
        out_shape=jax.ShapeDtypeStruct((1, _BS), jnp.float32),
    )(et, f1, w1t, c1, w2t, c2, wdt, cd)


def kernel(X_sparse, emb1, emb2, W1, b1, g1, be1, rm1, rv1,
           W2, b2, g2, be2, rm2, rv2, Wd, bd):
    idx = X_sparse.reshape(-1).astype(jnp.int32)
    p_tbl = emb2.reshape(_VOCAB // 8, 128)
    e1p = emb1.reshape(_VOCAB // 16, 16)
    s1 = g1 / jnp.sqrt(rv1 + 1e-5)
    w1t = (W1 * s1[None, :]).T
    c1 = ((b1 - rm1) * s1 + be1)[:, None]
    s2 = g2 / jnp.sqrt(rv2 + 1e-5)
    w2t = (W2 * s2[None, :]).T
    c2 = ((b2 - rm2) * s2 + be2)[:, None]
    wdt = Wd.T
    cd = bd[None, :]

    et, f1 = _sc_gather(idx, p_tbl, e1p)
    out = _tc_forward(et, f1.reshape(1, _BS), w1t, c1, w2t, c2, wdt, cd)
    return out.reshape(_BS, 1)
